# baseline (speedup 1.0000x reference)
import functools

import jax
import jax.numpy as jnp
from jax import lax
from jax.experimental import pallas as pl
from jax.experimental.pallas import tpu as pltpu
from jax.experimental.pallas import tpu_sc as plsc

VOCAB = 1_000_000
EMB = 32
UNITS = 64
BATCH = 16384
FIELDS = 26

BL = 65536
VOCAB_P = 16 * BL
GRID = VOCAB_P // BL

B_TOT = BATCH * FIELDS
NC, NS = 2, 16
NW = NC * NS
PER_W = B_TOT // NW


_DN1 = (((0,), (0,)), ((), ()))


def _table_body(et_ref, w1_ref, b1_ref, w2_ref, b2_ref, out_ref):
    e = jax.nn.relu(et_ref[...]).astype(jnp.bfloat16)
    h1 = lax.dot_general(w1_ref[...].astype(jnp.bfloat16), e, _DN1,
                         preferred_element_type=jnp.float32)
    h1 = jax.nn.relu(h1 + b1_ref[...])
    t = lax.dot_general(w2_ref[...], h1, _DN1,
                        preferred_element_type=jnp.float32)
    out_ref[...] = (t + b2_ref[...]).reshape(BL)


def _precompute_table(embT, w1, b1c, w2, b2c):
    return pl.pallas_call(
        _table_body,
        grid=(GRID,),
        in_specs=[
            pl.BlockSpec((EMB, BL), lambda i: (0, i)),
            pl.BlockSpec((EMB, UNITS), lambda i: (0, 0)),
            pl.BlockSpec((UNITS, 1), lambda i: (0, 0)),
            pl.BlockSpec((UNITS, 1), lambda i: (0, 0)),
            pl.BlockSpec((1, 1), lambda i: (0, 0)),
        ],
        out_specs=pl.BlockSpec((BL,), lambda i: (i,)),
        out_shape=jax.ShapeDtypeStruct((VOCAB_P,), jnp.float32),
    )(embT, w1, b1c, w2, b2c)


@functools.partial(
    pl.kernel,
    out_type=jax.ShapeDtypeStruct((B_TOT,), jnp.float32),
    mesh=plsc.VectorSubcoreMesh(core_axis_name="c", subcore_axis_name="s"),
    scratch_types=[
        pltpu.VMEM((PER_W,), jnp.int32),
        pltpu.VMEM((PER_W,), jnp.float32),
        pltpu.VMEM_SHARED((VOCAB_P,), jnp.float32),
        pltpu.SemaphoreType.DMA,
    ],
)
def _gather_scalars(table_hbm, idx_hbm, out_hbm, idx_v, vals_v, table_sp, sem):
    sid = lax.axis_index("s")
    wid = sid * NC + lax.axis_index("c")
    base = wid * PER_W
    CH = VOCAB_P // 16
    pltpu.sync_copy(table_hbm.at[pl.ds(sid * CH, CH)],
                    table_sp.at[pl.ds(sid * CH, CH)])
    pltpu.sync_copy(idx_hbm.at[pl.ds(base, PER_W)], idx_v)
    plsc.subcore_barrier()
    pltpu.async_copy(table_sp.at[idx_v], vals_v, sem).wait()
    pltpu.sync_copy(vals_v, out_hbm.at[pl.ds(base, PER_W)])


def kernel(x, emb, W1, b1, W2, b2):
    embT = emb.T
    b1c = b1[:, None]
    b2c = b2.reshape(1, 1)

    table = _precompute_table(embT, W1, b1c, W2, b2c)
    idx = x.astype(jnp.int32).T.reshape(B_TOT)
    out = _gather_scalars(table, idx)
    return out.reshape(FIELDS, BATCH, 1).transpose(1, 0, 2)


# speedup vs baseline: 1.0043x; 1.0043x over previous
import functools

import jax
import jax.numpy as jnp
from jax import lax
from jax.experimental import pallas as pl
from jax.experimental.pallas import tpu as pltpu
from jax.experimental.pallas import tpu_sc as plsc

VOCAB = 1_000_000
EMB = 32
UNITS = 64
BATCH = 16384
FIELDS = 26

BL = 65536
VOCAB_P = 16 * BL
GRID = VOCAB_P // BL

B_TOT = BATCH * FIELDS
NC, NS = 2, 16
NW = NC * NS
PER_W = B_TOT // NW


def _table_body(et_ref, w1t_ref, b1_ref, w2t_ref, b2_ref, out_ref):
    e = jax.nn.relu(et_ref[...]).astype(jnp.bfloat16)
    h1 = jnp.dot(w1t_ref[...].astype(jnp.bfloat16), e,
                 preferred_element_type=jnp.float32)
    h1 = jax.nn.relu(h1 + b1_ref[...])
    t = jnp.dot(w2t_ref[...], h1, preferred_element_type=jnp.float32)
    out_ref[...] = (t + b2_ref[...]).reshape(BL)


def _precompute_table(embT, w1t, b1c, w2t, b2c):
    return pl.pallas_call(
        _table_body,
        grid=(GRID,),
        in_specs=[
            pl.BlockSpec((EMB, BL), lambda i: (0, i)),
            pl.BlockSpec((UNITS, EMB), lambda i: (0, 0)),
            pl.BlockSpec((UNITS, 1), lambda i: (0, 0)),
            pl.BlockSpec((1, UNITS), lambda i: (0, 0)),
            pl.BlockSpec((1, 1), lambda i: (0, 0)),
        ],
        out_specs=pl.BlockSpec((BL,), lambda i: (i,)),
        out_shape=jax.ShapeDtypeStruct((VOCAB_P,), jnp.float32),
    )(embT, w1t, b1c, w2t, b2c)


@functools.partial(
    pl.kernel,
    out_type=jax.ShapeDtypeStruct((B_TOT,), jnp.float32),
    mesh=plsc.VectorSubcoreMesh(core_axis_name="c", subcore_axis_name="s"),
    scratch_types=[
        pltpu.VMEM((PER_W,), jnp.int32),
        pltpu.VMEM((PER_W,), jnp.float32),
        pltpu.VMEM_SHARED((VOCAB_P,), jnp.float32),
        pltpu.SemaphoreType.DMA,
    ],
)
def _gather_scalars(table_hbm, idx_hbm, out_hbm, idx_v, vals_v, table_sp, sem):
    sid = lax.axis_index("s")
    wid = sid * NC + lax.axis_index("c")
    base = wid * PER_W
    CH = VOCAB_P // 16
    pltpu.sync_copy(table_hbm.at[pl.ds(sid * CH, CH)],
                    table_sp.at[pl.ds(sid * CH, CH)])
    pltpu.sync_copy(idx_hbm.at[pl.ds(base, PER_W)], idx_v)
    plsc.subcore_barrier()
    pltpu.async_copy(table_sp.at[idx_v], vals_v, sem).wait()
    pltpu.sync_copy(vals_v, out_hbm.at[pl.ds(base, PER_W)])


def kernel(x, emb, W1, b1, W2, b2):
    embT = emb.T
    w1t = W1.T
    b1c = b1[:, None]
    w2t = W2.T
    b2c = b2.reshape(1, 1)

    table = _precompute_table(embT, w1t, b1c, w2t, b2c)
    idx = x.astype(jnp.int32).T.reshape(B_TOT)
    out = _gather_scalars(table, idx)
    return out.reshape(FIELDS, BATCH, 1).transpose(1, 0, 2)


# speedup vs baseline: 1.0045x; 1.0002x over previous
import functools

import jax
import jax.numpy as jnp
from jax import lax
from jax.experimental import pallas as pl
from jax.experimental.pallas import tpu as pltpu
from jax.experimental.pallas import tpu_sc as plsc

VOCAB = 1_000_000
EMB = 32
UNITS = 64
BATCH = 16384
FIELDS = 26

BL = 131072
VOCAB_P = 8 * BL
GRID = VOCAB_P // BL

B_TOT = BATCH * FIELDS
NC, NS = 2, 16
NW = NC * NS
PER_W = B_TOT // NW


def _table_body(et_ref, w1t_ref, b1_ref, w2t_ref, b2_ref, out_ref):
    e = jax.nn.relu(et_ref[...]).astype(jnp.bfloat16)
    h1 = jnp.dot(w1t_ref[...].astype(jnp.bfloat16), e,
                 preferred_element_type=jnp.float32)
    h1 = jax.nn.relu(h1 + b1_ref[...])
    t = jnp.dot(w2t_ref[...], h1, preferred_element_type=jnp.float32)
    out_ref[...] = (t + b2_ref[...]).reshape(BL)


def _precompute_table(embT, w1t, b1c, w2t, b2c):
    return pl.pallas_call(
        _table_body,
        grid=(GRID,),
        in_specs=[
            pl.BlockSpec((EMB, BL), lambda i: (0, i)),
            pl.BlockSpec((UNITS, EMB), lambda i: (0, 0)),
            pl.BlockSpec((UNITS, 1), lambda i: (0, 0)),
            pl.BlockSpec((1, UNITS), lambda i: (0, 0)),
            pl.BlockSpec((1, 1), lambda i: (0, 0)),
        ],
        out_specs=pl.BlockSpec((BL,), lambda i: (i,)),
        out_shape=jax.ShapeDtypeStruct((VOCAB_P,), jnp.float32),
    )(embT, w1t, b1c, w2t, b2c)


@functools.partial(
    pl.kernel,
    out_type=jax.ShapeDtypeStruct((B_TOT,), jnp.float32),
    mesh=plsc.VectorSubcoreMesh(core_axis_name="c", subcore_axis_name="s"),
    scratch_types=[
        pltpu.VMEM((PER_W,), jnp.int32),
        pltpu.VMEM((PER_W,), jnp.float32),
        pltpu.VMEM_SHARED((VOCAB_P,), jnp.float32),
        pltpu.SemaphoreType.DMA,
    ],
)
def _gather_scalars(table_hbm, idx_hbm, out_hbm, idx_v, vals_v, table_sp, sem):
    sid = lax.axis_index("s")
    wid = sid * NC + lax.axis_index("c")
    base = wid * PER_W
    CH = VOCAB_P // 16
    pltpu.sync_copy(table_hbm.at[pl.ds(sid * CH, CH)],
                    table_sp.at[pl.ds(sid * CH, CH)])
    pltpu.sync_copy(idx_hbm.at[pl.ds(base, PER_W)], idx_v)
    plsc.subcore_barrier()
    pltpu.async_copy(table_sp.at[idx_v], vals_v, sem).wait()
    pltpu.sync_copy(vals_v, out_hbm.at[pl.ds(base, PER_W)])


def kernel(x, emb, W1, b1, W2, b2):
    embT = emb.T
    w1t = W1.T
    b1c = b1[:, None]
    w2t = W2.T
    b2c = b2.reshape(1, 1)

    table = _precompute_table(embT, w1t, b1c, w2t, b2c)
    idx = x.astype(jnp.int32).T.reshape(B_TOT)
    out = _gather_scalars(table, idx)
    return out.reshape(FIELDS, BATCH, 1).transpose(1, 0, 2)


# speedup vs baseline: 1.0064x; 1.0019x over previous
import functools

import jax
import jax.numpy as jnp
from jax import lax
from jax.experimental import pallas as pl
from jax.experimental.pallas import tpu as pltpu
from jax.experimental.pallas import tpu_sc as plsc

VOCAB = 1_000_000
EMB = 32
UNITS = 64
BATCH = 16384
FIELDS = 26

BL = 65536
VOCAB_P = 16 * BL
GRID = VOCAB_P // BL

B_TOT = BATCH * FIELDS
NC, NS = 2, 16
NW = NC * NS
PER_W = B_TOT // NW


def _table_body(et_ref, w1t_ref, b1_ref, w2t_ref, b2_ref, out_ref):
    e = jax.nn.relu(et_ref[...]).astype(jnp.bfloat16)
    h1 = jnp.dot(w1t_ref[...].astype(jnp.bfloat16), e,
                 preferred_element_type=jnp.float32)
    h1 = jax.nn.relu(h1 + b1_ref[...])
    t = jnp.dot(w2t_ref[...], h1, preferred_element_type=jnp.float32)
    out_ref[...] = (t + b2_ref[...]).reshape(BL)


def _precompute_table(embT, w1t, b1c, w2t, b2c):
    return pl.pallas_call(
        _table_body,
        grid=(GRID,),
        in_specs=[
            pl.BlockSpec((EMB, BL), lambda i: (0, i)),
            pl.BlockSpec((UNITS, EMB), lambda i: (0, 0)),
            pl.BlockSpec((UNITS, 1), lambda i: (0, 0)),
            pl.BlockSpec((1, UNITS), lambda i: (0, 0)),
            pl.BlockSpec((1, 1), lambda i: (0, 0)),
        ],
        out_specs=pl.BlockSpec((BL,), lambda i: (i,)),
        out_shape=jax.ShapeDtypeStruct((VOCAB_P,), jnp.float32),
    )(embT, w1t, b1c, w2t, b2c)


@functools.partial(
    pl.kernel,
    out_type=jax.ShapeDtypeStruct((B_TOT,), jnp.float32),
    mesh=plsc.VectorSubcoreMesh(core_axis_name="c", subcore_axis_name="s"),
    scratch_types=[
        pltpu.VMEM((PER_W,), jnp.int32),
        pltpu.VMEM((PER_W,), jnp.float32),
        pltpu.VMEM_SHARED((VOCAB_P,), jnp.float32),
        pltpu.SemaphoreType.DMA,
    ],
)
def _gather_scalars(table_hbm, idx_hbm, out_hbm, idx_v, vals_v, table_sp, sem):
    sid = lax.axis_index("s")
    wid = sid * NC + lax.axis_index("c")
    base = wid * PER_W
    CH = VOCAB_P // 16
    pltpu.sync_copy(table_hbm.at[pl.ds(sid * CH, CH)],
                    table_sp.at[pl.ds(sid * CH, CH)])
    pltpu.sync_copy(idx_hbm.at[pl.ds(base, PER_W)], idx_v)
    plsc.subcore_barrier()
    pltpu.async_copy(table_sp.at[idx_v], vals_v, sem).wait()
    pltpu.sync_copy(vals_v, out_hbm.at[pl.ds(base, PER_W)])


def kernel(x, emb, W1, b1, W2, b2):
    embT = emb.T
    w1t = W1.T
    b1c = b1[:, None]
    w2t = W2.T
    b2c = b2.reshape(1, 1)

    table = _precompute_table(embT, w1t, b1c, w2t, b2c)
    idx = x.astype(jnp.int32).T.reshape(B_TOT)
    out = _gather_scalars(table, idx)
    return out.reshape(FIELDS, BATCH, 1).transpose(1, 0, 2)
